# baseline (speedup 1.0000x reference)
import functools

import jax
import jax.numpy as jnp
from jax import lax
from jax.experimental import pallas as pl
from jax.experimental.pallas import tpu as pltpu
from jax.experimental.pallas import tpu_sc as plsc

Q = 1024
D = 128
N_LBL = 100000
TOPK = 5
T = 2048
BIG_I = 2**30


def _inv_l2(ssum):
    x = jnp.maximum(ssum, 1e-37)
    y = lax.rsqrt(x)
    y = y * (1.5 - 0.5 * x * y * y)
    y = y * (1.5 - 0.5 * x * y * y)
    n = jnp.maximum(x * y, 1e-12)
    r = 1.0 / n
    r = r * (2.0 - n * r)
    r = r * (2.0 - n * r)
    return r


def _topk_tc_kernel(doc_ref, lbl_ref, scr_ref, idx_ref, bs_ref, bi_ref, cf_ref, dn_ref):
    t = pl.program_id(0)
    nt = pl.num_programs(0)

    @pl.when(t == 0)
    def _init():
        bs_ref[...] = jnp.full((Q, TOPK), -jnp.inf, jnp.float32)
        bi_ref[...] = jnp.full((Q, TOPK), float(BIG_I), jnp.float32)
        cf_ref[...] = lax.broadcasted_iota(jnp.int32, (Q, T), 1).astype(jnp.float32)
        doc = doc_ref[...]
        dn_ref[...] = doc * _inv_l2(jnp.sum(doc * doc, axis=1, keepdims=True))

    docn = dn_ref[...]

    lbl = lbl_ref[...]
    lbln = lbl * _inv_l2(jnp.sum(lbl * lbl, axis=1, keepdims=True))

    s = lax.dot_general(
        docn, lbln, (((1,), (1,)), ((), ())),
        preferred_element_type=jnp.float32,
    )

    cols = cf_ref[...]
    lim = (N_LBL - t * T).astype(jnp.float32)
    s = lax.cond(t == nt - 1,
                 lambda x: jnp.where(cols < lim, x, -jnp.inf),
                 lambda x: x, s)

    ts, ti = [], []
    for _ in range(TOPK):
        m = jnp.max(s, axis=1, keepdims=True)
        is_m = s == m
        pick = jnp.min(jnp.where(is_m, cols, jnp.inf), axis=1, keepdims=True)
        s = jnp.where(cols == pick, -jnp.inf, s)
        ts.append(m)
        ti.append(pick)
    tile_s = jnp.concatenate(ts, axis=1)
    tile_i = jnp.concatenate(ti, axis=1) + (t * T).astype(jnp.float32)

    cs = jnp.concatenate([bs_ref[...], tile_s], axis=1)
    ci = jnp.concatenate([bi_ref[...], tile_i], axis=1)
    ns, ni = [], []
    for _ in range(TOPK):
        m = jnp.max(cs, axis=1, keepdims=True)
        is_m = cs == m
        pick = jnp.min(jnp.where(is_m, ci, jnp.inf), axis=1, keepdims=True)
        cs = jnp.where(is_m & (ci == pick), -jnp.inf, cs)
        ns.append(m)
        ni.append(pick)
    bs_ref[...] = jnp.concatenate(ns, axis=1)
    bi_ref[...] = jnp.concatenate(ni, axis=1)

    @pl.when(t == nt - 1)
    def _fin():
        scr_ref[...] = bs_ref[...]
        idx_ref[...] = bi_ref[...].astype(jnp.int32)


def _topk_tc(doc, lbl_emb):
    nt = pl.cdiv(N_LBL, T)
    return pl.pallas_call(
        _topk_tc_kernel,
        grid=(nt,),
        in_specs=[
            pl.BlockSpec((Q, D), lambda t: (0, 0)),
            pl.BlockSpec((T, D), lambda t: (t, 0)),
        ],
        out_specs=[
            pl.BlockSpec((Q, TOPK), lambda t: (0, 0)),
            pl.BlockSpec((Q, TOPK), lambda t: (0, 0)),
        ],
        out_shape=[
            jax.ShapeDtypeStruct((Q, TOPK), jnp.float32),
            jax.ShapeDtypeStruct((Q, TOPK), jnp.int32),
        ],
        scratch_shapes=[
            pltpu.VMEM((Q, TOPK), jnp.float32),
            pltpu.VMEM((Q, TOPK), jnp.float32),
            pltpu.VMEM((Q, T), jnp.float32),
            pltpu.VMEM((Q, D), jnp.float32),
        ],
    )(doc, lbl_emb)


_B = Q * TOPK
_NW = 32
_PER_W = _B // _NW
_CH = 80
_NCH = _PER_W // _CH


def _remap_sc(ind_flat, table):
    mesh = plsc.VectorSubcoreMesh(core_axis_name="c", subcore_axis_name="s")

    @functools.partial(
        pl.kernel,
        mesh=mesh,
        out_type=jax.ShapeDtypeStruct((_B,), jnp.int32),
        scratch_types=[
            pltpu.VMEM((_CH,), jnp.int32),
            pltpu.VMEM((_CH,), jnp.int32),
            pltpu.SemaphoreType.DMA,
        ],
    )
    def k(idx_hbm, table_hbm, out_hbm, idx_v, rows_v, sem):
        wid = lax.axis_index("s") * 2 + lax.axis_index("c")
        for c in range(_NCH):
            base = wid * _PER_W + c * _CH
            pltpu.sync_copy(idx_hbm.at[pl.ds(base, _CH)], idx_v)
            pltpu.async_copy(table_hbm.at[idx_v], rows_v, sem).wait()
            pltpu.sync_copy(rows_v, out_hbm.at[pl.ds(base, _CH)])

    return k(ind_flat, table)


def kernel(doc, lbl_emb, remapped, K=5):
    scr, raw_idx = _topk_tc(doc, lbl_emb)
    ind = _remap_sc(raw_idx.reshape(-1), remapped).reshape(Q, TOPK)
    return (scr, ind)


# speedup vs baseline: 1.0728x; 1.0728x over previous
import functools

import jax
import jax.numpy as jnp
from jax import lax
from jax.experimental import pallas as pl
from jax.experimental.pallas import tpu as pltpu
from jax.experimental.pallas import tpu_sc as plsc

Q = 1024
D = 128
N_LBL = 100000
TOPK = 5
T = 2048
BIG_I = 2**30


def _inv_l2(ssum):
    x = jnp.maximum(ssum, 1e-37)
    y = lax.rsqrt(x)
    y = y * (1.5 - 0.5 * x * y * y)
    y = y * (1.5 - 0.5 * x * y * y)
    n = jnp.maximum(x * y, 1e-12)
    r = 1.0 / n
    r = r * (2.0 - n * r)
    r = r * (2.0 - n * r)
    return r


def _topk_tc_kernel(doc_ref, lbl_ref, scr_ref, idx_ref,
                    bs_ref, bi_ref, dn_ref, sw_ref, ts_ref, ti_ref):
    t = pl.program_id(0)
    nt = pl.num_programs(0)

    @pl.when(t == 0)
    def _init():
        bs_ref[...] = jnp.full((Q, TOPK), -jnp.inf, jnp.float32)
        bi_ref[...] = jnp.full((Q, TOPK), float(BIG_I), jnp.float32)
        doc = doc_ref[...]
        dn_ref[...] = doc * _inv_l2(jnp.sum(doc * doc, axis=1, keepdims=True))

    docn = dn_ref[...]

    lbl = lbl_ref[...]
    lbln = lbl * _inv_l2(jnp.sum(lbl * lbl, axis=1, keepdims=True))

    s = lax.dot_general(
        docn, lbln, (((1,), (1,)), ((), ())),
        preferred_element_type=jnp.float32,
    )

    lim = (N_LBL - t * T)
    ii = lax.broadcasted_iota(jnp.int32, (Q, T), 1)
    s = lax.cond(t == nt - 1,
                 lambda x: jnp.where(ii < lim, x, -jnp.inf),
                 lambda x: x, s)
    cols = ii.astype(jnp.float32)

    th = jnp.min(bs_ref[...], axis=1, keepdims=True)
    cnt = jnp.sum(jnp.where(s > th, 1.0, 0.0), axis=1, keepdims=True)
    npv = jnp.max(cnt)

    sw_ref[...] = s
    ts_ref[...] = jnp.full((Q, TOPK), -jnp.inf, jnp.float32)
    ti_ref[...] = jnp.full((Q, TOPK), float(BIG_I), jnp.float32)

    for j in range(TOPK):
        @pl.when(npv > j)
        def _pass(j=j):
            sv = sw_ref[...]
            m = jnp.max(sv, axis=1, keepdims=True)
            pick = jnp.min(jnp.where(sv == m, cols, jnp.inf), axis=1,
                           keepdims=True)
            sw_ref[...] = jnp.where(cols == pick, -jnp.inf, sv)
            ts_ref[:, j:j + 1] = m
            ti_ref[:, j:j + 1] = pick

    tile_s = ts_ref[...]
    tile_i = ti_ref[...] + (t * T).astype(jnp.float32)

    cs = jnp.concatenate([bs_ref[...], tile_s], axis=1)
    ci = jnp.concatenate([bi_ref[...], tile_i], axis=1)
    ns, ni = [], []
    for _ in range(TOPK):
        m = jnp.max(cs, axis=1, keepdims=True)
        is_m = cs == m
        pick = jnp.min(jnp.where(is_m, ci, jnp.inf), axis=1, keepdims=True)
        cs = jnp.where(is_m & (ci == pick), -jnp.inf, cs)
        ns.append(m)
        ni.append(pick)
    bs_ref[...] = jnp.concatenate(ns, axis=1)
    bi_ref[...] = jnp.concatenate(ni, axis=1)

    @pl.when(t == nt - 1)
    def _fin():
        scr_ref[...] = bs_ref[...]
        idx_ref[...] = bi_ref[...].astype(jnp.int32)


def _topk_tc(doc, lbl_emb):
    nt = pl.cdiv(N_LBL, T)
    return pl.pallas_call(
        _topk_tc_kernel,
        grid=(nt,),
        in_specs=[
            pl.BlockSpec((Q, D), lambda t: (0, 0)),
            pl.BlockSpec((T, D), lambda t: (t, 0)),
        ],
        out_specs=[
            pl.BlockSpec((Q, TOPK), lambda t: (0, 0)),
            pl.BlockSpec((Q, TOPK), lambda t: (0, 0)),
        ],
        out_shape=[
            jax.ShapeDtypeStruct((Q, TOPK), jnp.float32),
            jax.ShapeDtypeStruct((Q, TOPK), jnp.int32),
        ],
        scratch_shapes=[
            pltpu.VMEM((Q, TOPK), jnp.float32),
            pltpu.VMEM((Q, TOPK), jnp.float32),
            pltpu.VMEM((Q, D), jnp.float32),
            pltpu.VMEM((Q, T), jnp.float32),
            pltpu.VMEM((Q, TOPK), jnp.float32),
            pltpu.VMEM((Q, TOPK), jnp.float32),
        ],
    )(doc, lbl_emb)


_B = Q * TOPK
_NW = 32
_PER_W = _B // _NW
_CH = 80
_NCH = _PER_W // _CH


def _remap_sc(ind_flat, table):
    mesh = plsc.VectorSubcoreMesh(core_axis_name="c", subcore_axis_name="s")

    @functools.partial(
        pl.kernel,
        mesh=mesh,
        out_type=jax.ShapeDtypeStruct((_B,), jnp.int32),
        scratch_types=[
            pltpu.VMEM((_CH,), jnp.int32),
            pltpu.VMEM((_CH,), jnp.int32),
            pltpu.SemaphoreType.DMA,
        ],
    )
    def k(idx_hbm, table_hbm, out_hbm, idx_v, rows_v, sem):
        wid = lax.axis_index("s") * 2 + lax.axis_index("c")
        for c in range(_NCH):
            base = wid * _PER_W + c * _CH
            pltpu.sync_copy(idx_hbm.at[pl.ds(base, _CH)], idx_v)
            pltpu.async_copy(table_hbm.at[idx_v], rows_v, sem).wait()
            pltpu.sync_copy(rows_v, out_hbm.at[pl.ds(base, _CH)])

    return k(ind_flat, table)


def kernel(doc, lbl_emb, remapped, K=5):
    scr, raw_idx = _topk_tc(doc, lbl_emb)
    ind = _remap_sc(raw_idx.reshape(-1), remapped).reshape(Q, TOPK)
    return (scr, ind)
